# baseline (device time: 173745 ns/iter reference)
import jax
import jax.numpy as jnp
from jax import lax
from jax.experimental import pallas as pl
from jax.experimental.pallas import tpu as pltpu

NZ = 4


def kernel(O, Wo):
    B, S, Hs, D = O.shape
    K = Hs * D
    N = Wo.shape[1]
    S_out = S // NZ
    NSTEP = NZ - 1

    p = jnp.dot(
        O.reshape(B * S, K).astype(jnp.bfloat16),
        Wo.astype(jnp.bfloat16),
        preferred_element_type=jnp.bfloat16,
    ).reshape(B, S, N)

    def body(p_ref, out_ref, acc_ref, recv_ref, send_sems, recv_sems):
        x = lax.axis_index("x")
        y = lax.axis_index("y")
        z = lax.axis_index("z")
        right = (z + 1) % NZ
        left = (z + NZ - 1) % NZ

        barrier = pltpu.get_barrier_semaphore()
        for nbr in (left, right):
            pl.semaphore_signal(
                barrier, inc=1,
                device_id=(x, y, nbr),
                device_id_type=pl.DeviceIdType.MESH,
            )
        pl.semaphore_wait(barrier, 2)

        G = 4

        def mk(src, s, g):
            return pltpu.make_async_remote_copy(
                src_ref=src,
                dst_ref=recv_ref.at[s, g],
                send_sem=send_sems.at[s, g],
                recv_sem=recv_sems.at[s, g],
                device_id=(x, y, right),
                device_id_type=pl.DeviceIdType.MESH,
            )

        rdmas = {}
        c0 = (z + NZ - 1) % NZ
        for g in range(G):
            acc_ref[g] = p_ref[g, pl.ds(c0 * S_out, S_out), :]
            rdmas[(0, g)] = mk(acc_ref.at[g], 0, g)
            rdmas[(0, g)].start()

        for s in range(NSTEP):
            for g in range(G):
                rdmas[(s, g)].wait_recv()
                if s < NSTEP - 1:
                    rdmas[(s + 1, g)] = mk(recv_ref.at[s, g], s + 1, g)
                    rdmas[(s + 1, g)].start()
                else:
                    out_ref[g] = recv_ref[s, g].astype(jnp.float32)

        for s in range(NSTEP):
            for g in range(G):
                rdmas[(s, g)].wait_send()

    return pl.pallas_call(
        body,
        out_shape=jax.ShapeDtypeStruct((B, S_out, N), jnp.float32),
        in_specs=[pl.BlockSpec(memory_space=pltpu.VMEM)],
        out_specs=pl.BlockSpec(memory_space=pltpu.VMEM),
        scratch_shapes=[
            pltpu.VMEM((B, S_out, N), jnp.bfloat16),
            pltpu.VMEM((NSTEP, B, S_out, N), jnp.bfloat16),
            pltpu.SemaphoreType.DMA((NSTEP, 4)),
            pltpu.SemaphoreType.DMA((NSTEP, 4)),
        ],
        compiler_params=pltpu.CompilerParams(
            collective_id=0,
            vmem_limit_bytes=100 * 1024 * 1024,
        ),
    )(p)


# device time: 165045 ns/iter; 1.0527x vs baseline; 1.0527x over previous
import jax
import jax.numpy as jnp
from jax import lax
from jax.experimental import pallas as pl
from jax.experimental.pallas import tpu as pltpu

NZ = 4


def kernel(O, Wo):
    B, S, Hs, D = O.shape
    K = Hs * D
    N = Wo.shape[1]
    S_out = S // NZ
    NSTEP = NZ - 1

    O_bf = O.reshape(B, S, K).astype(jnp.bfloat16)
    Wo_bf = Wo.astype(jnp.bfloat16)

    def body(o_ref, w_ref, out_ref, acc_ref, pc_ref, recv_ref,
             send_sems, recv_sems):
        x = lax.axis_index("x")
        y = lax.axis_index("y")
        z = lax.axis_index("z")
        right = (z + 1) % NZ
        left = (z + NZ - 1) % NZ

        def pmm(c, b):
            return jnp.dot(
                o_ref[b, pl.ds(c * S_out, S_out), :], w_ref[...],
                preferred_element_type=jnp.float32,
            ).astype(jnp.bfloat16)

        def mk(s, b):
            return pltpu.make_async_remote_copy(
                src_ref=acc_ref.at[s, b],
                dst_ref=recv_ref.at[s, b],
                send_sem=send_sems.at[s, b],
                recv_sem=recv_sems.at[s, b],
                device_id=(x, y, right),
                device_id_type=pl.DeviceIdType.MESH,
            )

        c0 = (z + NZ - 1) % NZ
        for b in range(B):
            acc_ref[0, b] = pmm(c0, b)

        barrier = pltpu.get_barrier_semaphore()
        for nbr in (left, right):
            pl.semaphore_signal(
                barrier, inc=1,
                device_id=(x, y, nbr),
                device_id_type=pl.DeviceIdType.MESH,
            )
        pl.semaphore_wait(barrier, 2)

        rdmas = {}
        for b in range(B):
            rdmas[(0, b)] = mk(0, b)
            rdmas[(0, b)].start()

        for s in range(NSTEP):
            ridx = (z + NZ - 2 - s) % NZ
            for b in range(B):
                pc_ref[s, b] = pmm(ridx, b)

        for s in range(NSTEP):
            for b in range(B):
                rdmas[(s, b)].wait_recv()
                if s < NSTEP - 1:
                    acc_ref[s + 1, b] = recv_ref[s, b] + pc_ref[s, b]
                    rdmas[(s + 1, b)] = mk(s + 1, b)
                    rdmas[(s + 1, b)].start()
                else:
                    out_ref[b] = recv_ref[s, b] + pc_ref[s, b]

        for s in range(NSTEP):
            for b in range(B):
                rdmas[(s, b)].wait_send()

    return pl.pallas_call(
        body,
        out_shape=jax.ShapeDtypeStruct((B, S_out, N), jnp.bfloat16),
        in_specs=[
            pl.BlockSpec(memory_space=pltpu.VMEM),
            pl.BlockSpec(memory_space=pltpu.VMEM),
        ],
        out_specs=pl.BlockSpec(memory_space=pltpu.VMEM),
        scratch_shapes=[
            pltpu.VMEM((NSTEP, B, S_out, N), jnp.bfloat16),
            pltpu.VMEM((NSTEP, B, S_out, N), jnp.bfloat16),
            pltpu.VMEM((NSTEP, B, S_out, N), jnp.bfloat16),
            pltpu.SemaphoreType.DMA((NSTEP, B)),
            pltpu.SemaphoreType.DMA((NSTEP, B)),
        ],
        compiler_params=pltpu.CompilerParams(
            collective_id=0,
            vmem_limit_bytes=100 * 1024 * 1024,
        ),
    )(O_bf, Wo_bf)


# device time: 163766 ns/iter; 1.0609x vs baseline; 1.0078x over previous
import jax
import jax.numpy as jnp
from jax import lax
from jax.experimental import pallas as pl
from jax.experimental.pallas import tpu as pltpu

NZ = 4


def kernel(O, Wo):
    B, S, H, D = O.shape
    K = H * D
    N = Wo.shape[1]
    S_out = S // NZ
    NSTEP = NZ - 1

    O3 = O.reshape(B, S, K)

    def body(o_ref, w_ref, out_ref, wbf_ref, acc0_ref, pc_ref, recv_ref,
             send_sems, recv_sems):
        x = lax.axis_index("x")
        y = lax.axis_index("y")
        z = lax.axis_index("z")
        right = (z + 1) % NZ
        left = (z + NZ - 1) % NZ

        wbf_ref[...] = w_ref[...].astype(jnp.bfloat16)

        def pmm(c, b):
            return jnp.dot(
                o_ref[b, pl.ds(c * S_out, S_out), :].astype(jnp.bfloat16),
                wbf_ref[...],
                preferred_element_type=jnp.float32,
            ).astype(jnp.bfloat16)

        def mk(s, b):
            src = acc0_ref.at[b] if s == 0 else pc_ref.at[s - 1, b]
            return pltpu.make_async_remote_copy(
                src_ref=src,
                dst_ref=recv_ref.at[s, b],
                send_sem=send_sems.at[s, b],
                recv_sem=recv_sems.at[s, b],
                device_id=(x, y, right),
                device_id_type=pl.DeviceIdType.MESH,
            )

        c0 = (z + NZ - 1) % NZ
        for b in range(B):
            acc0_ref[b] = pmm(c0, b)

        barrier = pltpu.get_barrier_semaphore()
        for nbr in (left, right):
            pl.semaphore_signal(
                barrier, inc=1,
                device_id=(x, y, nbr),
                device_id_type=pl.DeviceIdType.MESH,
            )
        pl.semaphore_wait(barrier, 2)

        rdmas = {}
        for b in range(B):
            rdmas[(0, b)] = mk(0, b)
            rdmas[(0, b)].start()

        for s in range(NSTEP):
            ridx = (z + NZ - 2 - s) % NZ
            for b in range(B):
                pc_ref[s, b] = pmm(ridx, b)

        for s in range(NSTEP):
            for b in range(B):
                rdmas[(s, b)].wait_recv()
                if s < NSTEP - 1:
                    pc_ref[s, b] = pc_ref[s, b] + recv_ref[s, b]
                    rdmas[(s + 1, b)] = mk(s + 1, b)
                    rdmas[(s + 1, b)].start()
                else:
                    out_ref[b] = recv_ref[s, b] + pc_ref[s, b]

        for s in range(NSTEP):
            for b in range(B):
                rdmas[(s, b)].wait_send()

    return pl.pallas_call(
        body,
        out_shape=jax.ShapeDtypeStruct((B, S_out, N), jnp.bfloat16),
        in_specs=[
            pl.BlockSpec(memory_space=pltpu.VMEM),
            pl.BlockSpec(memory_space=pltpu.VMEM),
        ],
        out_specs=pl.BlockSpec(memory_space=pltpu.VMEM),
        scratch_shapes=[
            pltpu.VMEM((K, N), jnp.bfloat16),
            pltpu.VMEM((B, S_out, N), jnp.bfloat16),
            pltpu.VMEM((NSTEP, B, S_out, N), jnp.bfloat16),
            pltpu.VMEM((NSTEP, B, S_out, N), jnp.bfloat16),
            pltpu.SemaphoreType.DMA((NSTEP, B)),
            pltpu.SemaphoreType.DMA((NSTEP, B)),
        ],
        compiler_params=pltpu.CompilerParams(
            collective_id=0,
            vmem_limit_bytes=100 * 1024 * 1024,
        ),
    )(O3, Wo)


# device time: 159821 ns/iter; 1.0871x vs baseline; 1.0247x over previous
import jax
import jax.numpy as jnp
from jax import lax
from jax.experimental import pallas as pl
from jax.experimental.pallas import tpu as pltpu

NZ = 4


def kernel(O, Wo):
    B, S, H, D = O.shape
    K = H * D
    N = Wo.shape[1]
    S_out = S // NZ
    NSTEP = NZ - 1

    O3 = O.astype(jnp.bfloat16).reshape(B, S, K)

    def body(o_ref, w_ref, out_ref, wbf_ref, acc0_ref, pc_ref, recv_ref,
             send_sems, recv_sems):
        x = lax.axis_index("x")
        y = lax.axis_index("y")
        z = lax.axis_index("z")
        right = (z + 1) % NZ
        left = (z + NZ - 1) % NZ

        wbf_ref[...] = w_ref[...].astype(jnp.bfloat16)

        def pmm(c, b):
            return jnp.dot(
                o_ref[b, pl.ds(c * S_out, S_out), :],
                wbf_ref[...],
                preferred_element_type=jnp.float32,
            ).astype(jnp.bfloat16)

        def mk(s, b):
            src = acc0_ref.at[b] if s == 0 else pc_ref.at[s - 1, b]
            return pltpu.make_async_remote_copy(
                src_ref=src,
                dst_ref=recv_ref.at[s, b],
                send_sem=send_sems.at[s, b],
                recv_sem=recv_sems.at[s, b],
                device_id=(x, y, right),
                device_id_type=pl.DeviceIdType.MESH,
            )

        c0 = (z + NZ - 1) % NZ
        acc0_ref[0] = pmm(c0, 0)

        barrier = pltpu.get_barrier_semaphore()
        for nbr in (left, right):
            pl.semaphore_signal(
                barrier, inc=1,
                device_id=(x, y, nbr),
                device_id_type=pl.DeviceIdType.MESH,
            )
        pl.semaphore_wait(barrier, 2)

        rdmas = {}
        for b in range(B):
            if b > 0:
                acc0_ref[b] = pmm(c0, b)
            rdmas[(0, b)] = mk(0, b)
            rdmas[(0, b)].start()

        for s in range(NSTEP):
            ridx = (z + NZ - 2 - s) % NZ
            for b in range(B):
                pc_ref[s, b] = pmm(ridx, b)

        for s in range(NSTEP):
            for b in range(B):
                rdmas[(s, b)].wait_recv()
                if s < NSTEP - 1:
                    pc_ref[s, b] = pc_ref[s, b] + recv_ref[s, b]
                    rdmas[(s + 1, b)] = mk(s + 1, b)
                    rdmas[(s + 1, b)].start()
                else:
                    out_ref[b] = recv_ref[s, b] + pc_ref[s, b]

        for s in range(NSTEP):
            for b in range(B):
                rdmas[(s, b)].wait_send()

    return pl.pallas_call(
        body,
        out_shape=jax.ShapeDtypeStruct((B, S_out, N), jnp.bfloat16),
        in_specs=[
            pl.BlockSpec(memory_space=pltpu.VMEM),
            pl.BlockSpec(memory_space=pltpu.VMEM),
        ],
        out_specs=pl.BlockSpec(memory_space=pltpu.VMEM),
        scratch_shapes=[
            pltpu.VMEM((K, N), jnp.bfloat16),
            pltpu.VMEM((B, S_out, N), jnp.bfloat16),
            pltpu.VMEM((NSTEP, B, S_out, N), jnp.bfloat16),
            pltpu.VMEM((NSTEP, B, S_out, N), jnp.bfloat16),
            pltpu.SemaphoreType.DMA((NSTEP, B)),
            pltpu.SemaphoreType.DMA((NSTEP, B)),
        ],
        compiler_params=pltpu.CompilerParams(
            collective_id=0,
            vmem_limit_bytes=100 * 1024 * 1024,
        ),
    )(O3, Wo)
